# Initial kernel scaffold; baseline (speedup 1.0000x reference)
import jax
import jax.numpy as jnp
from jax.experimental import pallas as pl


def kernel(x, start_core, end_core, cores):
    raise NotImplementedError("write your pallas kernel here")


# speedup vs baseline: 5.1351x; 5.1351x over previous
import functools

import jax
import jax.numpy as jnp
from jax import lax
from jax.experimental import pallas as pl
from jax.experimental.pallas import tpu as pltpu
from jax.experimental.pallas import tpu_sc as plsc

B = 16384
HR = 100
RANK = 16
DIMC = 4
NPAIR = HR * HR
ROWW = 256

NC = 2
NS = 16
NW = NC * NS
PER_W = B // NW
CH = 128
NCHUNK = PER_W // CH
L = 16

H1B = 25
TC_GRID = HR // H1B


def _pair_kernel(ct_ref, e_ref, o_ref):
    e = e_ref[...]
    for i in range(H1B):
        c = ct_ref[i]
        o_ref[i] = jnp.dot(e, c, precision=lax.Precision.HIGHEST,
                           preferred_element_type=jnp.float32)


def _pair_table(ct, e2):
    return pl.pallas_call(
        _pair_kernel,
        grid=(TC_GRID,),
        in_specs=[
            pl.BlockSpec((H1B, RANK, 64), lambda i: (i, 0, 0)),
            pl.BlockSpec((4 * HR, RANK), lambda i: (0, 0)),
        ],
        out_specs=pl.BlockSpec((H1B, 4 * HR, 64), lambda i: (i, 0, 0)),
        out_shape=jax.ShapeDtypeStruct((HR, 4 * HR, 64), jnp.float32),
    )(ct, e2)


_MESH = plsc.VectorSubcoreMesh(core_axis_name="c", subcore_axis_name="s")


@functools.partial(
    pl.kernel,
    out_type=jax.ShapeDtypeStruct((B * 64,), jnp.float32),
    mesh=_MESH,
    compiler_params=pltpu.CompilerParams(needs_layout_passes=False),
    scratch_types=[
        pltpu.VMEM((PER_W,), jnp.int32),
        pltpu.VMEM((CH,), jnp.int32),
        pltpu.VMEM((HR * 64,), jnp.float32),
        pltpu.VMEM((CH, ROWW), jnp.float32),
        pltpu.VMEM((CH * 64,), jnp.float32),
        pltpu.SemaphoreType.DMA,
    ],
)
def _sc_lookup(x_hbm, t_hbm, s_hbm, out_hbm, x_v, idx_v, s_v, trows_v, ob_v, sem):
    wid = lax.axis_index("s") * NC + lax.axis_index("c")
    base = wid * PER_W
    pltpu.sync_copy(s_hbm, s_v)
    pltpu.sync_copy(x_hbm.at[pl.ds(base, PER_W)], x_v)

    lanes = lax.iota(jnp.int32, L)

    def chunk_body(c, carry):
        cbase = c * CH
        c100 = jnp.full((L,), 100, jnp.int32)
        c10000 = jnp.full((L,), 10000, jnp.int32)
        for i in range(CH // L):
            xv = x_v[pl.ds(cbase + i * L, L)]
            h1 = lax.rem(lax.div(xv, c100), c100)
            h2 = lax.div(xv, c10000)
            idx_v[pl.ds(i * L, L)] = h1 * 100 + h2

        pltpu.async_copy(t_hbm.at[idx_v], trows_v, sem).wait()

        def group_body(g, carry2):
            xg = x_v[pl.ds(cbase + g * L, L)]
            sbase = lax.rem(xg, c100) * 64
            rowg = g * L + lanes
            for d in range(DIMC):
                accs = [[None] * DIMC for _ in range(DIMC)]
                for s in range(RANK):
                    svals = [plsc.load_gather(s_v, [sbase + (D * 16 + s)])
                             for D in range(DIMC)]
                    for e in range(DIMC):
                        pos = e * 64 + d * 16 + s
                        tval = plsc.load_gather(
                            trows_v, [rowg, jnp.full((L,), pos, jnp.int32)])
                        for D in range(DIMC):
                            prod = svals[D] * tval
                            accs[D][e] = prod if s == 0 else accs[D][e] + prod
                obase = (g * L + lanes) * 64
                for D in range(DIMC):
                    for e in range(DIMC):
                        j = D * 16 + d * 4 + e
                        plsc.store_scatter(ob_v, [obase + j], accs[D][e])
            return carry2
        lax.fori_loop(0, CH // L, group_body, 0)

        pltpu.sync_copy(ob_v, out_hbm.at[pl.ds((base + cbase) * 64, CH * 64)])
        return carry

    lax.fori_loop(0, NCHUNK, chunk_body, 0)


def kernel(x, start_core, end_core, cores):
    ct = jnp.transpose(cores[0].reshape(HR, 64, RANK), (0, 2, 1))
    e2 = end_core.reshape(4 * HR, RANK)
    t = _pair_table(ct, e2).reshape(NPAIR, ROWW)
    s_flat = start_core.reshape(HR * 64)
    out = _sc_lookup(x.astype(jnp.int32), t, s_flat)
    return out.reshape(B, 64)


# speedup vs baseline: 8.9607x; 1.7450x over previous
import functools

import jax
import jax.numpy as jnp
from jax import lax
from jax.experimental import pallas as pl
from jax.experimental.pallas import tpu as pltpu
from jax.experimental.pallas import tpu_sc as plsc

B = 16384
HR = 100
RANK = 16
DIMC = 4
NPAIR = HR * HR
ROWW = 256

NC = 2
NS = 16
NW = NC * NS
PER_W = B // NW
CH = 128
NCHUNK = PER_W // CH
L = 16

H1B = 25
TC_GRID = HR // H1B


def _pair_kernel(ct_ref, e_ref, o_ref):
    e = e_ref[...]
    for i in range(H1B):
        c = ct_ref[i]
        o_ref[i] = jnp.dot(e, c, precision=lax.Precision.HIGHEST,
                           preferred_element_type=jnp.float32)


def _pair_table(ct, e2):
    return pl.pallas_call(
        _pair_kernel,
        grid=(TC_GRID,),
        in_specs=[
            pl.BlockSpec((H1B, RANK, 64), lambda i: (i, 0, 0)),
            pl.BlockSpec((4 * HR, RANK), lambda i: (0, 0)),
        ],
        out_specs=pl.BlockSpec((H1B, 4 * HR, 64), lambda i: (i, 0, 0)),
        out_shape=jax.ShapeDtypeStruct((HR, 4 * HR, 64), jnp.float32),
    )(ct, e2)


_MESH = plsc.VectorSubcoreMesh(core_axis_name="c", subcore_axis_name="s")


@functools.partial(
    pl.kernel,
    out_type=jax.ShapeDtypeStruct((B * 64,), jnp.float32),
    mesh=_MESH,
    compiler_params=pltpu.CompilerParams(needs_layout_passes=False),
    scratch_types=[
        pltpu.VMEM((PER_W,), jnp.int32),
        pltpu.VMEM((CH,), jnp.int32),
        pltpu.VMEM((HR * 64,), jnp.float32),
        pltpu.VMEM((CH, ROWW), jnp.float32),
        pltpu.VMEM((CH * 65,), jnp.float32),
        pltpu.VMEM((CH * 64,), jnp.float32),
        pltpu.SemaphoreType.DMA,
    ],
)
def _sc_lookup(x_hbm, t_hbm, s_hbm, out_hbm, x_v, idx_v, s_v, trows_v, ob2_v,
               ob_v, sem):
    wid = lax.axis_index("s") * NC + lax.axis_index("c")
    base = wid * PER_W
    pltpu.sync_copy(s_hbm, s_v)
    pltpu.sync_copy(x_hbm.at[pl.ds(base, PER_W)], x_v)

    lanes = lax.iota(jnp.int32, L)

    def chunk_body(c, carry):
        cbase = c * CH
        c100 = jnp.full((L,), 100, jnp.int32)
        c10000 = jnp.full((L,), 10000, jnp.int32)
        for i in range(CH // L):
            xv = x_v[pl.ds(cbase + i * L, L)]
            h1 = lax.rem(lax.div(xv, c100), c100)
            h2 = lax.div(xv, c10000)
            idx_v[pl.ds(i * L, L)] = h1 * 100 + h2

        pltpu.async_copy(t_hbm.at[idx_v], trows_v, sem).wait()

        def group_body(g, carry2):
            xg = x_v[pl.ds(cbase + g * L, L)]
            sbase = lax.rem(xg, c100) * 64
            rowg = g * L + lanes
            obase = (g * L + lanes) * 65
            rots = [(lanes + t) & 15 for t in range(RANK)]
            sb = [sbase + D * 16 for D in range(DIMC)]
            for d in range(DIMC):
                accs = [[None] * DIMC for _ in range(DIMC)]
                for t in range(RANK):
                    rot = rots[t]
                    svals = [plsc.load_gather(s_v, [sb[D] + rot])
                             for D in range(DIMC)]
                    for e in range(DIMC):
                        tval = plsc.load_gather(
                            trows_v, [rowg, (e * 64 + d * 16) + rot])
                        for D in range(DIMC):
                            prod = svals[D] * tval
                            accs[D][e] = prod if t == 0 else accs[D][e] + prod
                for D in range(DIMC):
                    for e in range(DIMC):
                        j = D * 16 + d * 4 + e
                        plsc.store_scatter(ob2_v, [obase + j], accs[D][e])
            return carry2
        lax.fori_loop(0, CH // L, group_body, 0)

        def comp_body(i, carry2):
            for r in range(4):
                b = i * 4 + r
                for k in range(4):
                    ob_v[pl.ds(b * 64 + k * L, L)] = \
                        ob2_v[pl.ds(b * 65 + k * L, L)]
            return carry2
        lax.fori_loop(0, CH // 4, comp_body, 0)

        pltpu.sync_copy(ob_v, out_hbm.at[pl.ds((base + cbase) * 64, CH * 64)])
        return carry

    lax.fori_loop(0, NCHUNK, chunk_body, 0)


def kernel(x, start_core, end_core, cores):
    ct = jnp.transpose(cores[0].reshape(HR, 64, RANK), (0, 2, 1))
    e2 = end_core.reshape(4 * HR, RANK)
    t = _pair_table(ct, e2).reshape(NPAIR, ROWW)
    s_flat = start_core.reshape(HR * 64)
    out = _sc_lookup(x.astype(jnp.int32), t, s_flat)
    return out.reshape(B, 64)


# speedup vs baseline: 10.1602x; 1.1339x over previous
import functools

import jax
import jax.numpy as jnp
from jax import lax
from jax.experimental import pallas as pl
from jax.experimental.pallas import tpu as pltpu
from jax.experimental.pallas import tpu_sc as plsc

B = 16384
HR = 100
RANK = 16
DIMC = 4
NPAIR = HR * HR
ROWW = 256

NC = 2
NS = 16
NW = NC * NS
PER_W = B // NW
CH = 128
NCHUNK = PER_W // CH
L = 16

H1B = 50
TC_GRID = HR // H1B


def _pair_kernel(ct_ref, e_ref, o_ref):
    e = e_ref[...]
    for i in range(H1B):
        c = ct_ref[i]
        o_ref[i] = jnp.dot(e, c, preferred_element_type=jnp.float32)


def _pair_table(ct, e2):
    return pl.pallas_call(
        _pair_kernel,
        grid=(TC_GRID,),
        in_specs=[
            pl.BlockSpec((H1B, RANK, 64), lambda i: (i, 0, 0)),
            pl.BlockSpec((4 * HR, RANK), lambda i: (0, 0)),
        ],
        out_specs=pl.BlockSpec((H1B, 4 * HR, 64), lambda i: (i, 0, 0)),
        out_shape=jax.ShapeDtypeStruct((HR, 4 * HR, 64), jnp.float32),
    )(ct, e2)


_MESH = plsc.VectorSubcoreMesh(core_axis_name="c", subcore_axis_name="s")


@functools.partial(
    pl.kernel,
    out_type=jax.ShapeDtypeStruct((B, 64), jnp.float32),
    mesh=_MESH,
    compiler_params=pltpu.CompilerParams(needs_layout_passes=False),
    scratch_types=[
        pltpu.VMEM((PER_W,), jnp.int32),
        pltpu.VMEM((CH,), jnp.int32),
        pltpu.VMEM((HR * 64,), jnp.float32),
        pltpu.VMEM((CH, ROWW), jnp.float32),
        pltpu.VMEM((CH * 65,), jnp.float32),
        pltpu.VMEM((CH, 64), jnp.float32),
        pltpu.SemaphoreType.DMA,
    ],
)
def _sc_lookup(x_hbm, t_hbm, s_hbm, out_hbm, x_v, idx_v, s_v, trows_v, ob2_v,
               ob_v, sem):
    wid = lax.axis_index("s") * NC + lax.axis_index("c")
    base = wid * PER_W
    pltpu.sync_copy(s_hbm, s_v)
    pltpu.sync_copy(x_hbm.at[pl.ds(base, PER_W)], x_v)

    lanes = lax.iota(jnp.int32, L)

    def chunk_body(c, carry):
        cbase = c * CH
        c100 = jnp.full((L,), 100, jnp.int32)
        c10000 = jnp.full((L,), 10000, jnp.int32)
        for i in range(CH // L):
            xv = x_v[pl.ds(cbase + i * L, L)]
            h1 = lax.rem(lax.div(xv, c100), c100)
            h2 = lax.div(xv, c10000)
            idx_v[pl.ds(i * L, L)] = h1 * 100 + h2

        pltpu.async_copy(t_hbm.at[idx_v], trows_v, sem).wait()

        def group_body(g, carry2):
            xg = x_v[pl.ds(cbase + g * L, L)]
            sbase = lax.rem(xg, c100) * 64
            rowg = g * L + lanes
            obase = (g * L + lanes) * 65
            rots = [(lanes + t) & 15 for t in range(RANK)]
            sb = [sbase + D * 16 for D in range(DIMC)]
            for d in range(DIMC):
                accs = [[None] * DIMC for _ in range(DIMC)]
                for t in range(RANK):
                    rot = rots[t]
                    svals = [plsc.load_gather(s_v, [sb[D] + rot])
                             for D in range(DIMC)]
                    for e in range(DIMC):
                        tval = plsc.load_gather(
                            trows_v, [rowg, (e * 64 + d * 16) + rot])
                        for D in range(DIMC):
                            prod = svals[D] * tval
                            accs[D][e] = prod if t == 0 else accs[D][e] + prod
                for D in range(DIMC):
                    for e in range(DIMC):
                        j = D * 16 + d * 4 + e
                        plsc.store_scatter(ob2_v, [obase + j], accs[D][e])
            return carry2
        lax.fori_loop(0, CH // L, group_body, 0)

        def comp_body(i, carry2):
            for r in range(4):
                b = i * 4 + r
                for k in range(4):
                    ob_v[b, pl.ds(k * L, L)] = ob2_v[pl.ds(b * 65 + k * L, L)]
            return carry2
        lax.fori_loop(0, CH // 4, comp_body, 0)

        pltpu.sync_copy(ob_v, out_hbm.at[pl.ds(base + cbase, CH)])
        return carry

    lax.fori_loop(0, NCHUNK, chunk_body, 0)


def kernel(x, start_core, end_core, cores):
    ct = jnp.transpose(cores[0].reshape(HR, 64, RANK), (0, 2, 1))
    e2 = end_core.reshape(4 * HR, RANK)
    t = _pair_table(ct, e2).reshape(NPAIR, ROWW)
    s_flat = start_core.reshape(HR * 64)
    return _sc_lookup(x.astype(jnp.int32), t, s_flat)
